# baseline (speedup 1.0000x reference)
import functools

import jax
import jax.numpy as jnp
from jax import lax
from jax.experimental import pallas as pl
from jax.experimental.pallas import tpu as pltpu
from jax.experimental.pallas import tpu_sc as plsc

N = 2048
D = 768
F = 3072
E = 8
EP = 128
K = 2
A = N * K
CH = 128
NCH = A // CH
TM = 256
PMAX = ((A + E * (TM - 1)) + TM - 1) // TM * TM
NT = PMAX // TM
FB = 512
NJ = F // FB
NW = 32
TOK_W = N // NW


def _router_body(x_ref, wr_ref, pos_ref, w0_ref, w1_ref, cnt_ref, aux_ref):
    x = x_ref[...]
    wr = wr_ref[...]
    logits = lax.dot_general(x, wr, (((1,), (1,)), ((), ())),
                             preferred_element_type=jnp.float32)
    lane = lax.broadcasted_iota(jnp.int32, (N, EP), 1)
    valid = lane < E
    logits = jnp.where(valid, logits, -1e30)
    m = jnp.max(logits, axis=1, keepdims=True)
    ex = jnp.exp(logits - m)
    ex = jnp.where(valid, ex, 0.0)
    denom = jnp.sum(ex, axis=1, keepdims=True)
    probs = ex / denom

    p1 = jnp.max(probs, axis=1, keepdims=True)
    i1 = jnp.min(jnp.where(probs == p1, lane, EP), axis=1, keepdims=True)
    probs2 = jnp.where(lane == i1, -1.0, probs)
    p2 = jnp.max(probs2, axis=1, keepdims=True)
    i2 = jnp.min(jnp.where(probs2 == p2, lane, EP), axis=1, keepdims=True)
    s = p1 + p2
    w0_ref[...] = p1 / s
    w1_ref[...] = p2 / s

    e1 = (lane == i1).astype(jnp.float32)
    e2 = (lane == i2).astype(jnp.float32)
    onehot = jnp.concatenate([e1, e2], axis=0)

    r = lax.broadcasted_iota(jnp.int32, (CH, CH), 0)
    c = lax.broadcasted_iota(jnp.int32, (CH, CH), 1)
    ls = (r > c).astype(jnp.float32)
    us = (r < c).astype(jnp.float32)
    cums = []
    tots = []
    for ci in range(NCH):
        oc = onehot[ci * CH:(ci + 1) * CH, :]
        cums.append(lax.dot_general(ls, oc, (((1,), (0,)), ((), ())),
                                    preferred_element_type=jnp.float32))
        tots.append(jnp.sum(oc, axis=0, keepdims=True))
    t = jnp.concatenate(tots, axis=0)
    r32 = lax.broadcasted_iota(jnp.int32, (NCH, NCH), 0)
    c32 = lax.broadcasted_iota(jnp.int32, (NCH, NCH), 1)
    ls32 = (r32 > c32).astype(jnp.float32)
    off = lax.dot_general(ls32, t, (((1,), (0,)), ((), ())),
                          preferred_element_type=jnp.float32)
    counts = jnp.sum(t, axis=0, keepdims=True)
    padded = jnp.floor((counts + (TM - 1)) / TM) * TM
    base = lax.dot_general(padded, us, (((1,), (0,)), ((), ())),
                           preferred_element_type=jnp.float32)

    poss = []
    for ci in range(NCH):
        oc = onehot[ci * CH:(ci + 1) * CH, :]
        pc = cums[ci] + off[ci:ci + 1, :] + base
        poss.append(jnp.sum(oc * pc, axis=1, keepdims=True))
    pos = jnp.concatenate(poss, axis=0)
    pos_ref[...] = pos.astype(jnp.int32)
    cnt_ref[...] = counts.astype(jnp.int32)

    importance = jnp.sum(probs, axis=0, keepdims=True)
    aux = jnp.sum(importance * counts) * (E / (N * N + 1e-06))
    aux_ref[...] = jnp.broadcast_to(aux, (1, 1))


def _router(x_flat, wr_pad):
    return pl.pallas_call(
        _router_body,
        out_shape=(
            jax.ShapeDtypeStruct((A, 1), jnp.int32),
            jax.ShapeDtypeStruct((N, 1), jnp.float32),
            jax.ShapeDtypeStruct((N, 1), jnp.float32),
            jax.ShapeDtypeStruct((1, EP), jnp.int32),
            jax.ShapeDtypeStruct((1, 1), jnp.float32),
        ),
    )(x_flat, wr_pad)


def _dispatch_body(x_hbm, p0_hbm, p1_hbm, xs_hbm, i0_v, i1_v, xbuf, sem):
    wid = lax.axis_index("s") * 2 + lax.axis_index("c")
    t0 = wid * TOK_W
    pltpu.sync_copy(p0_hbm.at[pl.ds(t0, TOK_W)], i0_v)
    pltpu.sync_copy(p1_hbm.at[pl.ds(t0, TOK_W)], i1_v)
    pltpu.sync_copy(x_hbm.at[pl.ds(t0, TOK_W)], xbuf)
    pltpu.async_copy(xbuf, xs_hbm.at[i0_v], sem).wait()
    pltpu.async_copy(xbuf, xs_hbm.at[i1_v], sem).wait()


def _dispatch(x_flat, p0, p1):
    mesh = plsc.VectorSubcoreMesh(core_axis_name="c", subcore_axis_name="s")
    return pl.kernel(
        _dispatch_body,
        out_type=jax.ShapeDtypeStruct((PMAX, D // 2), jnp.int32),
        mesh=mesh,
        scratch_types=[
            pltpu.VMEM((TOK_W,), jnp.int32),
            pltpu.VMEM((TOK_W,), jnp.int32),
            pltpu.VMEM((TOK_W, D // 2), jnp.int32),
            pltpu.SemaphoreType.DMA,
        ],
    )(x_flat, p0, p1)


def _ffn_body(te_ref, xs_ref, w1_ref, w3_ref, w2_ref, out_ref,
              w1b_ref, w3b_ref, w2b_ref):
    j = pl.program_id(0)
    i = pl.program_id(1)
    prev = te_ref[jnp.maximum(i - 1, 0)]
    fresh = jnp.logical_or(i == 0, te_ref[i] != prev)

    @pl.when(fresh)
    def _():
        w1b_ref[...] = w1_ref[0].astype(jnp.bfloat16)
        w3b_ref[...] = w3_ref[0].astype(jnp.bfloat16)
        w2b_ref[...] = w2_ref[0].astype(jnp.bfloat16)

    x = xs_ref[...]
    a = lax.dot_general(x, w1b_ref[...], (((1,), (1,)), ((), ())),
                        preferred_element_type=jnp.float32)
    b = lax.dot_general(x, w3b_ref[...], (((1,), (1,)), ((), ())),
                        preferred_element_type=jnp.float32)
    h = (a * (1.0 / (1.0 + jnp.exp(-a))) * b).astype(jnp.bfloat16)
    o = lax.dot_general(h, w2b_ref[...], (((1,), (1,)), ((), ())),
                        preferred_element_type=jnp.float32)
    rows = pl.ds(i * TM, TM)

    @pl.when(j == 0)
    def _():
        out_ref[rows, :] = o

    @pl.when(j > 0)
    def _():
        out_ref[rows, :] += o


def _ffn(tile_expert, xs, w1, w2, w3):
    grid_spec = pltpu.PrefetchScalarGridSpec(
        num_scalar_prefetch=1,
        grid=(NJ, NT),
        in_specs=[
            pl.BlockSpec((TM, D), lambda j, i, te: (i, 0)),
            pl.BlockSpec((1, FB, D), lambda j, i, te: (te[i], j, 0)),
            pl.BlockSpec((1, FB, D), lambda j, i, te: (te[i], j, 0)),
            pl.BlockSpec((1, D, FB), lambda j, i, te: (te[i], 0, j)),
        ],
        out_specs=pl.BlockSpec((PMAX, D), lambda j, i, te: (0, 0)),
        scratch_shapes=[
            pltpu.VMEM((FB, D), jnp.bfloat16),
            pltpu.VMEM((FB, D), jnp.bfloat16),
            pltpu.VMEM((D, FB), jnp.bfloat16),
        ],
    )
    return pl.pallas_call(
        _ffn_body,
        grid_spec=grid_spec,
        out_shape=jax.ShapeDtypeStruct((PMAX, D), jnp.float32),
        compiler_params=pltpu.CompilerParams(
            dimension_semantics=("arbitrary", "arbitrary"),
            vmem_limit_bytes=112 * 1024 * 1024,
        ),
    )(tile_expert, xs, w1, w3, w2)


def _combine_body(y_hbm, p0_hbm, p1_hbm, wv0_hbm, wv1_hbm, out_hbm,
                  i0_v, i1_v, wv0_v, wv1_v, buf0, buf1, sem):
    wid = lax.axis_index("s") * 2 + lax.axis_index("c")
    t0 = wid * TOK_W
    pltpu.sync_copy(p0_hbm.at[pl.ds(t0, TOK_W)], i0_v)
    pltpu.sync_copy(p1_hbm.at[pl.ds(t0, TOK_W)], i1_v)
    pltpu.sync_copy(wv0_hbm.at[pl.ds(t0, TOK_W)], wv0_v)
    pltpu.sync_copy(wv1_hbm.at[pl.ds(t0, TOK_W)], wv1_v)
    c0 = pltpu.async_copy(y_hbm.at[i0_v], buf0, sem)
    c1 = pltpu.async_copy(y_hbm.at[i1_v], buf1, sem)
    c0.wait()
    c1.wait()

    def tok_body(i, _):
        lanes = jnp.zeros((16,), jnp.int32) + i
        w0b = plsc.load_gather(wv0_v, [lanes])
        w1b = plsc.load_gather(wv1_v, [lanes])
        for cc in range(D // 16):
            v = (w0b * buf0[i, pl.ds(cc * 16, 16)]
                 + w1b * buf1[i, pl.ds(cc * 16, 16)])
            buf0[i, pl.ds(cc * 16, 16)] = v
        return 0

    lax.fori_loop(0, TOK_W, tok_body, 0)
    pltpu.sync_copy(buf0, out_hbm.at[pl.ds(t0, TOK_W)])


def _combine(y, p0, p1, w0, w1):
    mesh = plsc.VectorSubcoreMesh(core_axis_name="c", subcore_axis_name="s")
    return pl.kernel(
        _combine_body,
        out_type=jax.ShapeDtypeStruct((N, D), jnp.float32),
        mesh=mesh,
        scratch_types=[
            pltpu.VMEM((TOK_W,), jnp.int32),
            pltpu.VMEM((TOK_W,), jnp.int32),
            pltpu.VMEM((TOK_W,), jnp.float32),
            pltpu.VMEM((TOK_W,), jnp.float32),
            pltpu.VMEM((TOK_W, D), jnp.float32),
            pltpu.VMEM((TOK_W, D), jnp.float32),
            pltpu.SemaphoreType.DMA,
        ],
        compiler_params=pltpu.CompilerParams(needs_layout_passes=False),
    )(y, p0, p1, w0, w1)


def kernel(x, Wr, W1, W2, W3):
    Bx, Sx, Dx = x.shape
    x_flat = x.reshape(N, D)
    wr_pad = jnp.concatenate(
        [Wr, jnp.zeros((EP - E, D), jnp.float32)], axis=0)

    pos, w0, w1, cnt, aux = _router(x_flat, wr_pad)
    pos_flat = pos.reshape(A)
    p0 = pos_flat[:N]
    p1 = pos_flat[N:]
    counts = cnt[0, :E]

    tiles_e = (counts + (TM - 1)) // TM
    ends = jnp.cumsum(tiles_e)
    ti = jnp.arange(NT, dtype=jnp.int32)
    tile_expert = jnp.minimum(
        jnp.sum((ti[:, None] >= ends[None, :]).astype(jnp.int32), axis=1),
        E - 1).astype(jnp.int32)

    x_words = lax.bitcast_convert_type(
        x_flat.astype(jnp.bfloat16).reshape(N, D // 2, 2), jnp.int32)
    xs_words = _dispatch(x_words, p0, p1)
    xs = lax.bitcast_convert_type(
        xs_words, jnp.bfloat16).reshape(PMAX, D)
    y = _ffn(tile_expert, xs, W1, W2, W3)
    out = _combine(y, p0, p1, w0.reshape(N), w1.reshape(N))
    return out.reshape(Bx, Sx, Dx), aux[0, 0]


# speedup vs baseline: 1.4249x; 1.4249x over previous
import functools

import jax
import jax.numpy as jnp
from jax import lax
from jax.experimental import pallas as pl
from jax.experimental.pallas import tpu as pltpu
from jax.experimental.pallas import tpu_sc as plsc

N = 2048
D = 768
F = 3072
E = 8
EP = 128
K = 2
A = N * K
CH = 128
NCH = A // CH
TM = 256
PMAX = ((A + E * (TM - 1)) + TM - 1) // TM * TM
NT = PMAX // TM
FB = 512
NJ = F // FB
NW = 32
TOK_W = N // NW


def _router_body(x_ref, wr_ref, pos_ref, w0_ref, w1_ref, cnt_ref, aux_ref):
    x = x_ref[...]
    wr = wr_ref[...]
    logits = lax.dot_general(x, wr, (((1,), (1,)), ((), ())),
                             preferred_element_type=jnp.float32)
    lane = lax.broadcasted_iota(jnp.int32, (N, EP), 1)
    valid = lane < E
    logits = jnp.where(valid, logits, -1e30)
    m = jnp.max(logits, axis=1, keepdims=True)
    ex = jnp.exp(logits - m)
    ex = jnp.where(valid, ex, 0.0)
    denom = jnp.sum(ex, axis=1, keepdims=True)
    probs = ex / denom

    p1 = jnp.max(probs, axis=1, keepdims=True)
    i1 = jnp.min(jnp.where(probs == p1, lane, EP), axis=1, keepdims=True)
    probs2 = jnp.where(lane == i1, -1.0, probs)
    p2 = jnp.max(probs2, axis=1, keepdims=True)
    i2 = jnp.min(jnp.where(probs2 == p2, lane, EP), axis=1, keepdims=True)
    s = p1 + p2
    w0_ref[...] = p1 / s
    w1_ref[...] = p2 / s

    e1 = (lane == i1).astype(jnp.float32)
    e2 = (lane == i2).astype(jnp.float32)
    onehot = jnp.concatenate([e1, e2], axis=0)

    r = lax.broadcasted_iota(jnp.int32, (CH, CH), 0)
    c = lax.broadcasted_iota(jnp.int32, (CH, CH), 1)
    ls = (r > c).astype(jnp.float32)
    us = (r < c).astype(jnp.float32)
    cums = []
    tots = []
    for ci in range(NCH):
        oc = onehot[ci * CH:(ci + 1) * CH, :]
        cums.append(lax.dot_general(ls, oc, (((1,), (0,)), ((), ())),
                                    preferred_element_type=jnp.float32))
        tots.append(jnp.sum(oc, axis=0, keepdims=True))
    t = jnp.concatenate(tots, axis=0)
    r32 = lax.broadcasted_iota(jnp.int32, (NCH, NCH), 0)
    c32 = lax.broadcasted_iota(jnp.int32, (NCH, NCH), 1)
    ls32 = (r32 > c32).astype(jnp.float32)
    off = lax.dot_general(ls32, t, (((1,), (0,)), ((), ())),
                          preferred_element_type=jnp.float32)
    counts = jnp.sum(t, axis=0, keepdims=True)
    padded = jnp.floor((counts + (TM - 1)) / TM) * TM
    base = lax.dot_general(padded, us, (((1,), (0,)), ((), ())),
                           preferred_element_type=jnp.float32)

    poss = []
    for ci in range(NCH):
        oc = onehot[ci * CH:(ci + 1) * CH, :]
        pc = cums[ci] + off[ci:ci + 1, :] + base
        poss.append(jnp.sum(oc * pc, axis=1, keepdims=True))
    pos = jnp.concatenate(poss, axis=0)
    pos_ref[...] = pos.astype(jnp.int32)
    cnt_ref[...] = counts.astype(jnp.int32)

    importance = jnp.sum(probs, axis=0, keepdims=True)
    aux = jnp.sum(importance * counts) * (E / (N * N + 1e-06))
    aux_ref[...] = jnp.broadcast_to(aux, (1, 1))


def _router(x_flat, wr_pad):
    return pl.pallas_call(
        _router_body,
        out_shape=(
            jax.ShapeDtypeStruct((A, 1), jnp.int32),
            jax.ShapeDtypeStruct((N, 1), jnp.float32),
            jax.ShapeDtypeStruct((N, 1), jnp.float32),
            jax.ShapeDtypeStruct((1, EP), jnp.int32),
            jax.ShapeDtypeStruct((1, 1), jnp.float32),
        ),
    )(x_flat, wr_pad)


def _dispatch_body(x_hbm, p0_hbm, p1_hbm, xs_hbm, i0_v, i1_v, xbuf, sem):
    wid = lax.axis_index("s") * 2 + lax.axis_index("c")
    t0 = wid * TOK_W
    pltpu.sync_copy(p0_hbm.at[pl.ds(t0, TOK_W)], i0_v)
    pltpu.sync_copy(p1_hbm.at[pl.ds(t0, TOK_W)], i1_v)
    pltpu.sync_copy(x_hbm.at[pl.ds(t0, TOK_W)], xbuf)
    pltpu.async_copy(xbuf, xs_hbm.at[i0_v], sem).wait()
    pltpu.async_copy(xbuf, xs_hbm.at[i1_v], sem).wait()


def _dispatch(x_flat, p0, p1):
    mesh = plsc.VectorSubcoreMesh(core_axis_name="c", subcore_axis_name="s")
    return pl.kernel(
        _dispatch_body,
        out_type=jax.ShapeDtypeStruct((PMAX, D), jnp.float32),
        mesh=mesh,
        scratch_types=[
            pltpu.VMEM((TOK_W,), jnp.int32),
            pltpu.VMEM((TOK_W,), jnp.int32),
            pltpu.VMEM((TOK_W, D), jnp.float32),
            pltpu.SemaphoreType.DMA,
        ],
    )(x_flat, p0, p1)


def _ffn_body(te_ref, xs_ref, w1_ref, w3_ref, w2_ref, out_ref,
              w1b_ref, w3b_ref, w2b_ref):
    j = pl.program_id(0)
    i = pl.program_id(1)
    prev = te_ref[jnp.maximum(i - 1, 0)]
    fresh = jnp.logical_or(i == 0, te_ref[i] != prev)

    @pl.when(fresh)
    def _():
        w1b_ref[...] = w1_ref[0].astype(jnp.bfloat16)
        w3b_ref[...] = w3_ref[0].astype(jnp.bfloat16)
        w2b_ref[...] = w2_ref[0].astype(jnp.bfloat16)

    x = xs_ref[...].astype(jnp.bfloat16)
    a = lax.dot_general(x, w1b_ref[...], (((1,), (1,)), ((), ())),
                        preferred_element_type=jnp.float32)
    b = lax.dot_general(x, w3b_ref[...], (((1,), (1,)), ((), ())),
                        preferred_element_type=jnp.float32)
    h = (a * (1.0 / (1.0 + jnp.exp(-a))) * b).astype(jnp.bfloat16)
    o = lax.dot_general(h, w2b_ref[...], (((1,), (1,)), ((), ())),
                        preferred_element_type=jnp.float32)
    rows = pl.ds(i * TM, TM)

    @pl.when(j == 0)
    def _():
        out_ref[rows, :] = o

    @pl.when(j > 0)
    def _():
        out_ref[rows, :] += o


def _ffn(tile_expert, xs, w1, w2, w3):
    grid_spec = pltpu.PrefetchScalarGridSpec(
        num_scalar_prefetch=1,
        grid=(NJ, NT),
        in_specs=[
            pl.BlockSpec((TM, D), lambda j, i, te: (i, 0)),
            pl.BlockSpec((1, FB, D), lambda j, i, te: (te[i], j, 0)),
            pl.BlockSpec((1, FB, D), lambda j, i, te: (te[i], j, 0)),
            pl.BlockSpec((1, D, FB), lambda j, i, te: (te[i], 0, j)),
        ],
        out_specs=pl.BlockSpec((PMAX, D), lambda j, i, te: (0, 0)),
        scratch_shapes=[
            pltpu.VMEM((FB, D), jnp.bfloat16),
            pltpu.VMEM((FB, D), jnp.bfloat16),
            pltpu.VMEM((D, FB), jnp.bfloat16),
        ],
    )
    return pl.pallas_call(
        _ffn_body,
        grid_spec=grid_spec,
        out_shape=jax.ShapeDtypeStruct((PMAX, D), jnp.float32),
        compiler_params=pltpu.CompilerParams(
            dimension_semantics=("arbitrary", "arbitrary"),
            vmem_limit_bytes=112 * 1024 * 1024,
        ),
    )(tile_expert, xs, w1, w3, w2)


def _combine_body(y_hbm, p0_hbm, p1_hbm, wv0_hbm, wv1_hbm, out_hbm,
                  i0_v, i1_v, wv0_v, wv1_v, buf0, buf1, sem):
    wid = lax.axis_index("s") * 2 + lax.axis_index("c")
    t0 = wid * TOK_W
    pltpu.sync_copy(p0_hbm.at[pl.ds(t0, TOK_W)], i0_v)
    pltpu.sync_copy(p1_hbm.at[pl.ds(t0, TOK_W)], i1_v)
    pltpu.sync_copy(wv0_hbm.at[pl.ds(t0, TOK_W)], wv0_v)
    pltpu.sync_copy(wv1_hbm.at[pl.ds(t0, TOK_W)], wv1_v)
    c0 = pltpu.async_copy(y_hbm.at[i0_v], buf0, sem)
    c1 = pltpu.async_copy(y_hbm.at[i1_v], buf1, sem)
    c0.wait()
    c1.wait()

    def tok_body(i, _):
        lanes = jnp.zeros((16,), jnp.int32) + i
        w0b = plsc.load_gather(wv0_v, [lanes])
        w1b = plsc.load_gather(wv1_v, [lanes])
        for cc in range(D // 16):
            v = (w0b * buf0[i, pl.ds(cc * 16, 16)]
                 + w1b * buf1[i, pl.ds(cc * 16, 16)])
            buf0[i, pl.ds(cc * 16, 16)] = v
        return 0

    lax.fori_loop(0, TOK_W, tok_body, 0)
    pltpu.sync_copy(buf0, out_hbm.at[pl.ds(t0, TOK_W)])


def _combine(y, p0, p1, w0, w1):
    mesh = plsc.VectorSubcoreMesh(core_axis_name="c", subcore_axis_name="s")
    return pl.kernel(
        _combine_body,
        out_type=jax.ShapeDtypeStruct((N, D), jnp.float32),
        mesh=mesh,
        scratch_types=[
            pltpu.VMEM((TOK_W,), jnp.int32),
            pltpu.VMEM((TOK_W,), jnp.int32),
            pltpu.VMEM((TOK_W,), jnp.float32),
            pltpu.VMEM((TOK_W,), jnp.float32),
            pltpu.VMEM((TOK_W, D), jnp.float32),
            pltpu.VMEM((TOK_W, D), jnp.float32),
            pltpu.SemaphoreType.DMA,
        ],
        compiler_params=pltpu.CompilerParams(needs_layout_passes=False),
    )(y, p0, p1, w0, w1)


def kernel(x, Wr, W1, W2, W3):
    Bx, Sx, Dx = x.shape
    x_flat = x.reshape(N, D)
    wr_pad = jnp.concatenate(
        [Wr, jnp.zeros((EP - E, D), jnp.float32)], axis=0)

    pos, w0, w1, cnt, aux = _router(x_flat, wr_pad)
    pos_flat = pos.reshape(A)
    p0 = pos_flat[:N]
    p1 = pos_flat[N:]
    counts = cnt[0, :E]

    tiles_e = (counts + (TM - 1)) // TM
    ends = jnp.cumsum(tiles_e)
    ti = jnp.arange(NT, dtype=jnp.int32)
    tile_expert = jnp.minimum(
        jnp.sum((ti[:, None] >= ends[None, :]).astype(jnp.int32), axis=1),
        E - 1).astype(jnp.int32)

    xs = _dispatch(x_flat, p0, p1)
    y = _ffn(tile_expert, xs, W1, W2, W3)
    out = _combine(y, p0, p1, w0.reshape(N), w1.reshape(N))
    return out.reshape(Bx, Sx, Dx), aux[0, 0]


# speedup vs baseline: 1.8610x; 1.3061x over previous
import functools

import jax
import jax.numpy as jnp
from jax import lax
from jax.experimental import pallas as pl
from jax.experimental.pallas import tpu as pltpu
from jax.experimental.pallas import tpu_sc as plsc

N = 2048
D = 768
F = 3072
E = 8
EP = 128
K = 2
A = N * K
CH = 128
NCH = A // CH
TM = 256
PMAX = ((A + E * (TM - 1)) + TM - 1) // TM * TM
NT = PMAX // TM
FB = 1024
NJ = F // FB
NW = 32
TOK_W = N // NW


def _router_body(x_ref, wr_ref, pos_ref, w0_ref, w1_ref, cnt_ref, aux_ref):
    x = x_ref[...]
    wr = wr_ref[...]
    logits = lax.dot_general(x, wr, (((1,), (1,)), ((), ())),
                             preferred_element_type=jnp.float32)
    lane = lax.broadcasted_iota(jnp.int32, (N, EP), 1)
    valid = lane < E
    logits = jnp.where(valid, logits, -1e30)
    m = jnp.max(logits, axis=1, keepdims=True)
    ex = jnp.exp(logits - m)
    ex = jnp.where(valid, ex, 0.0)
    denom = jnp.sum(ex, axis=1, keepdims=True)
    probs = ex / denom

    p1 = jnp.max(probs, axis=1, keepdims=True)
    i1 = jnp.min(jnp.where(probs == p1, lane, EP), axis=1, keepdims=True)
    probs2 = jnp.where(lane == i1, -1.0, probs)
    p2 = jnp.max(probs2, axis=1, keepdims=True)
    i2 = jnp.min(jnp.where(probs2 == p2, lane, EP), axis=1, keepdims=True)
    s = p1 + p2
    w0_ref[...] = p1 / s
    w1_ref[...] = p2 / s

    e1 = (lane == i1).astype(jnp.float32)
    e2 = (lane == i2).astype(jnp.float32)
    onehot = jnp.concatenate([e1, e2], axis=0)

    r = lax.broadcasted_iota(jnp.int32, (CH, CH), 0)
    c = lax.broadcasted_iota(jnp.int32, (CH, CH), 1)
    ls = (r > c).astype(jnp.float32)
    us = (r < c).astype(jnp.float32)
    cums = []
    tots = []
    for ci in range(NCH):
        oc = onehot[ci * CH:(ci + 1) * CH, :]
        cums.append(lax.dot_general(ls, oc, (((1,), (0,)), ((), ())),
                                    preferred_element_type=jnp.float32))
        tots.append(jnp.sum(oc, axis=0, keepdims=True))
    t = jnp.concatenate(tots, axis=0)
    r32 = lax.broadcasted_iota(jnp.int32, (NCH, NCH), 0)
    c32 = lax.broadcasted_iota(jnp.int32, (NCH, NCH), 1)
    ls32 = (r32 > c32).astype(jnp.float32)
    off = lax.dot_general(ls32, t, (((1,), (0,)), ((), ())),
                          preferred_element_type=jnp.float32)
    counts = jnp.sum(t, axis=0, keepdims=True)
    padded = jnp.floor((counts + (TM - 1)) / TM) * TM
    base = lax.dot_general(padded, us, (((1,), (0,)), ((), ())),
                           preferred_element_type=jnp.float32)

    poss = []
    for ci in range(NCH):
        oc = onehot[ci * CH:(ci + 1) * CH, :]
        pc = cums[ci] + off[ci:ci + 1, :] + base
        poss.append(jnp.sum(oc * pc, axis=1, keepdims=True))
    pos = jnp.concatenate(poss, axis=0)
    pos_ref[...] = pos.astype(jnp.int32)
    cnt_ref[...] = counts.astype(jnp.int32)

    importance = jnp.sum(probs, axis=0, keepdims=True)
    aux = jnp.sum(importance * counts) * (E / (N * N + 1e-06))
    aux_ref[...] = jnp.broadcast_to(aux, (1, 1))


def _router(x_flat, wr_pad):
    return pl.pallas_call(
        _router_body,
        out_shape=(
            jax.ShapeDtypeStruct((A, 1), jnp.int32),
            jax.ShapeDtypeStruct((N, 1), jnp.float32),
            jax.ShapeDtypeStruct((N, 1), jnp.float32),
            jax.ShapeDtypeStruct((1, EP), jnp.int32),
            jax.ShapeDtypeStruct((1, 1), jnp.float32),
        ),
    )(x_flat, wr_pad)


def _dispatch_body(x_hbm, p0_hbm, p1_hbm, xs_hbm, i0_v, i1_v, xbuf, sem):
    wid = lax.axis_index("s") * 2 + lax.axis_index("c")
    t0 = wid * TOK_W
    pltpu.sync_copy(p0_hbm.at[pl.ds(t0, TOK_W)], i0_v)
    pltpu.sync_copy(p1_hbm.at[pl.ds(t0, TOK_W)], i1_v)
    pltpu.sync_copy(x_hbm.at[pl.ds(t0, TOK_W)], xbuf)
    pltpu.async_copy(xbuf, xs_hbm.at[i0_v], sem).wait()
    pltpu.async_copy(xbuf, xs_hbm.at[i1_v], sem).wait()


def _dispatch(x_flat, p0, p1):
    mesh = plsc.VectorSubcoreMesh(core_axis_name="c", subcore_axis_name="s")
    return pl.kernel(
        _dispatch_body,
        out_type=jax.ShapeDtypeStruct((PMAX, D), jnp.float32),
        mesh=mesh,
        scratch_types=[
            pltpu.VMEM((TOK_W,), jnp.int32),
            pltpu.VMEM((TOK_W,), jnp.int32),
            pltpu.VMEM((TOK_W, D), jnp.float32),
            pltpu.SemaphoreType.DMA,
        ],
    )(x_flat, p0, p1)


def _ffn_body(te_ref, xs_ref, w1_ref, w3_ref, w2_ref, out_ref,
              w1b_ref, w3b_ref, w2b_ref):
    j = pl.program_id(0)
    i = pl.program_id(1)
    n_used = te_ref[NT]

    @pl.when(i < n_used)
    def _():
        prev = te_ref[jnp.maximum(i - 1, 0)]
        fresh = jnp.logical_or(i == 0, te_ref[i] != prev)

        @pl.when(fresh)
        def _():
            w1b_ref[...] = w1_ref[0].astype(jnp.bfloat16)
            w3b_ref[...] = w3_ref[0].astype(jnp.bfloat16)
            w2b_ref[...] = w2_ref[0].astype(jnp.bfloat16)

        x = xs_ref[...].astype(jnp.bfloat16)
        a = lax.dot_general(x, w1b_ref[...], (((1,), (1,)), ((), ())),
                            preferred_element_type=jnp.float32)
        b = lax.dot_general(x, w3b_ref[...], (((1,), (1,)), ((), ())),
                            preferred_element_type=jnp.float32)
        h = (a * (1.0 / (1.0 + jnp.exp(-a))) * b).astype(jnp.bfloat16)
        o = lax.dot_general(h, w2b_ref[...], (((1,), (1,)), ((), ())),
                            preferred_element_type=jnp.float32)
        rows = pl.ds(i * TM, TM)

        @pl.when(j == 0)
        def _():
            out_ref[rows, :] = o

        @pl.when(j > 0)
        def _():
            out_ref[rows, :] += o


def _ffn(tile_expert, xs, w1, w2, w3):
    def iclamp(i, te):
        return jnp.minimum(i, te[NT] - 1)

    grid_spec = pltpu.PrefetchScalarGridSpec(
        num_scalar_prefetch=1,
        grid=(NJ, NT),
        in_specs=[
            pl.BlockSpec((TM, D), lambda j, i, te: (iclamp(i, te), 0)),
            pl.BlockSpec((1, FB, D), lambda j, i, te: (te[iclamp(i, te)], j, 0)),
            pl.BlockSpec((1, FB, D), lambda j, i, te: (te[iclamp(i, te)], j, 0)),
            pl.BlockSpec((1, D, FB), lambda j, i, te: (te[iclamp(i, te)], 0, j)),
        ],
        out_specs=pl.BlockSpec((PMAX, D), lambda j, i, te: (0, 0)),
        scratch_shapes=[
            pltpu.VMEM((FB, D), jnp.bfloat16),
            pltpu.VMEM((FB, D), jnp.bfloat16),
            pltpu.VMEM((D, FB), jnp.bfloat16),
        ],
    )
    return pl.pallas_call(
        _ffn_body,
        grid_spec=grid_spec,
        out_shape=jax.ShapeDtypeStruct((PMAX, D), jnp.float32),
        compiler_params=pltpu.CompilerParams(
            dimension_semantics=("arbitrary", "arbitrary"),
            vmem_limit_bytes=60 * 1024 * 1024,
        ),
    )(tile_expert, xs, w1, w3, w2)


def _combine_body(y_hbm, p0_hbm, p1_hbm, wv0_hbm, wv1_hbm, out_hbm,
                  i0_v, i1_v, wv0_v, wv1_v, buf0, buf1, sem):
    wid = lax.axis_index("s") * 2 + lax.axis_index("c")
    t0 = wid * TOK_W
    pltpu.sync_copy(p0_hbm.at[pl.ds(t0, TOK_W)], i0_v)
    pltpu.sync_copy(p1_hbm.at[pl.ds(t0, TOK_W)], i1_v)
    pltpu.sync_copy(wv0_hbm.at[pl.ds(t0, TOK_W)], wv0_v)
    pltpu.sync_copy(wv1_hbm.at[pl.ds(t0, TOK_W)], wv1_v)
    c0 = pltpu.async_copy(y_hbm.at[i0_v], buf0, sem)
    c1 = pltpu.async_copy(y_hbm.at[i1_v], buf1, sem)
    c0.wait()
    c1.wait()

    def tok_body(i, _):
        lanes = jnp.zeros((16,), jnp.int32) + i
        w0b = plsc.load_gather(wv0_v, [lanes])
        w1b = plsc.load_gather(wv1_v, [lanes])
        for cc in range(D // 16):
            v = (w0b * buf0[i, pl.ds(cc * 16, 16)]
                 + w1b * buf1[i, pl.ds(cc * 16, 16)])
            buf0[i, pl.ds(cc * 16, 16)] = v
        return 0

    lax.fori_loop(0, TOK_W, tok_body, 0)
    pltpu.sync_copy(buf0, out_hbm.at[pl.ds(t0, TOK_W)])


def _combine(y, p0, p1, w0, w1):
    mesh = plsc.VectorSubcoreMesh(core_axis_name="c", subcore_axis_name="s")
    return pl.kernel(
        _combine_body,
        out_type=jax.ShapeDtypeStruct((N, D), jnp.float32),
        mesh=mesh,
        scratch_types=[
            pltpu.VMEM((TOK_W,), jnp.int32),
            pltpu.VMEM((TOK_W,), jnp.int32),
            pltpu.VMEM((TOK_W,), jnp.float32),
            pltpu.VMEM((TOK_W,), jnp.float32),
            pltpu.VMEM((TOK_W, D), jnp.float32),
            pltpu.VMEM((TOK_W, D), jnp.float32),
            pltpu.SemaphoreType.DMA,
        ],
        compiler_params=pltpu.CompilerParams(needs_layout_passes=False),
    )(y, p0, p1, w0, w1)


def kernel(x, Wr, W1, W2, W3):
    Bx, Sx, Dx = x.shape
    x_flat = x.reshape(N, D)
    wr_pad = jnp.concatenate(
        [Wr, jnp.zeros((EP - E, D), jnp.float32)], axis=0)

    pos, w0, w1, cnt, aux = _router(x_flat, wr_pad)
    pos_flat = pos.reshape(A)
    p0 = pos_flat[:N]
    p1 = pos_flat[N:]
    counts = cnt[0, :E]

    tiles_e = (counts + (TM - 1)) // TM
    ends = jnp.cumsum(tiles_e)
    ti = jnp.arange(NT, dtype=jnp.int32)
    tile_expert = jnp.minimum(
        jnp.sum((ti[:, None] >= ends[None, :]).astype(jnp.int32), axis=1),
        E - 1).astype(jnp.int32)
    tile_expert = jnp.concatenate(
        [tile_expert, ends[E - 1:E].astype(jnp.int32)])

    xs = _dispatch(x_flat, p0, p1)
    y = _ffn(tile_expert, xs, W1, W2, W3)
    out = _combine(y, p0, p1, w0.reshape(N), w1.reshape(N))
    return out.reshape(Bx, Sx, Dx), aux[0, 0]


# speedup vs baseline: 1.9576x; 1.0519x over previous
import functools

import jax
import jax.numpy as jnp
from jax import lax
from jax.experimental import pallas as pl
from jax.experimental.pallas import tpu as pltpu
from jax.experimental.pallas import tpu_sc as plsc

N = 2048
D = 768
F = 3072
E = 8
EP = 128
K = 2
A = N * K
CH = 128
NCH = A // CH
TM = 256
PMAX = ((A + E * (TM - 1)) + TM - 1) // TM * TM
NT = PMAX // TM
FB = 1024
NJ = F // FB
NW = 32
TOK_W = N // NW


def _router_body(x_ref, wr_ref, pos_ref, w0_ref, w1_ref, cnt_ref, aux_ref):
    x = x_ref[...]
    wr = wr_ref[...]
    logits = lax.dot_general(x, wr, (((1,), (1,)), ((), ())),
                             preferred_element_type=jnp.float32)
    lane = lax.broadcasted_iota(jnp.int32, (N, EP), 1)
    valid = lane < E
    logits = jnp.where(valid, logits, -1e30)
    m = jnp.max(logits, axis=1, keepdims=True)
    ex = jnp.exp(logits - m)
    ex = jnp.where(valid, ex, 0.0)
    denom = jnp.sum(ex, axis=1, keepdims=True)
    probs = ex / denom

    p1 = jnp.max(probs, axis=1, keepdims=True)
    i1 = jnp.min(jnp.where(probs == p1, lane, EP), axis=1, keepdims=True)
    probs2 = jnp.where(lane == i1, -1.0, probs)
    p2 = jnp.max(probs2, axis=1, keepdims=True)
    i2 = jnp.min(jnp.where(probs2 == p2, lane, EP), axis=1, keepdims=True)
    s = p1 + p2
    w0_ref[...] = p1 / s
    w1_ref[...] = p2 / s

    e1 = (lane == i1).astype(jnp.float32)
    e2 = (lane == i2).astype(jnp.float32)
    onehot = jnp.concatenate([e1, e2], axis=0)

    r = lax.broadcasted_iota(jnp.int32, (CH, CH), 0)
    c = lax.broadcasted_iota(jnp.int32, (CH, CH), 1)
    ls = (r > c).astype(jnp.float32)
    us = (r < c).astype(jnp.float32)
    cums = []
    tots = []
    for ci in range(NCH):
        oc = onehot[ci * CH:(ci + 1) * CH, :]
        cums.append(lax.dot_general(ls, oc, (((1,), (0,)), ((), ())),
                                    preferred_element_type=jnp.float32))
        tots.append(jnp.sum(oc, axis=0, keepdims=True))
    t = jnp.concatenate(tots, axis=0)
    r32 = lax.broadcasted_iota(jnp.int32, (NCH, NCH), 0)
    c32 = lax.broadcasted_iota(jnp.int32, (NCH, NCH), 1)
    ls32 = (r32 > c32).astype(jnp.float32)
    off = lax.dot_general(ls32, t, (((1,), (0,)), ((), ())),
                          preferred_element_type=jnp.float32)
    counts = jnp.sum(t, axis=0, keepdims=True)
    padded = jnp.floor((counts + (TM - 1)) / TM) * TM
    base = lax.dot_general(padded, us, (((1,), (0,)), ((), ())),
                           preferred_element_type=jnp.float32)

    poss = []
    for ci in range(NCH):
        oc = onehot[ci * CH:(ci + 1) * CH, :]
        pc = cums[ci] + off[ci:ci + 1, :] + base
        poss.append(jnp.sum(oc * pc, axis=1, keepdims=True))
    pos = jnp.concatenate(poss, axis=0)
    pos_ref[...] = pos.astype(jnp.int32)
    cnt_ref[...] = counts.astype(jnp.int32)

    importance = jnp.sum(probs, axis=0, keepdims=True)
    aux = jnp.sum(importance * counts) * (E / (N * N + 1e-06))
    aux_ref[...] = jnp.broadcast_to(aux, (1, 1))


def _router(x_flat, wr_pad):
    return pl.pallas_call(
        _router_body,
        out_shape=(
            jax.ShapeDtypeStruct((A, 1), jnp.int32),
            jax.ShapeDtypeStruct((N, 1), jnp.float32),
            jax.ShapeDtypeStruct((N, 1), jnp.float32),
            jax.ShapeDtypeStruct((1, EP), jnp.int32),
            jax.ShapeDtypeStruct((1, 1), jnp.float32),
        ),
    )(x_flat, wr_pad)


def _dispatch_body(x_hbm, p0_hbm, p1_hbm, xs_hbm, i0_v, i1_v, xbuf, sem):
    wid = lax.axis_index("s") * 2 + lax.axis_index("c")
    t0 = wid * TOK_W
    pltpu.sync_copy(p0_hbm.at[pl.ds(t0, TOK_W)], i0_v)
    pltpu.sync_copy(p1_hbm.at[pl.ds(t0, TOK_W)], i1_v)
    pltpu.sync_copy(x_hbm.at[pl.ds(t0, TOK_W)], xbuf)
    pltpu.async_copy(xbuf, xs_hbm.at[i0_v], sem).wait()
    pltpu.async_copy(xbuf, xs_hbm.at[i1_v], sem).wait()


def _dispatch(x_flat, p0, p1):
    mesh = plsc.VectorSubcoreMesh(core_axis_name="c", subcore_axis_name="s")
    return pl.kernel(
        _dispatch_body,
        out_type=jax.ShapeDtypeStruct((PMAX, D), jnp.float32),
        mesh=mesh,
        scratch_types=[
            pltpu.VMEM((TOK_W,), jnp.int32),
            pltpu.VMEM((TOK_W,), jnp.int32),
            pltpu.VMEM((TOK_W, D), jnp.float32),
            pltpu.SemaphoreType.DMA,
        ],
    )(x_flat, p0, p1)


def _ffn_body(te_ref, xs_ref, w1_ref, w3_ref, w2_ref, out_ref):
    j = pl.program_id(0)
    i = pl.program_id(1)
    n_used = te_ref[NT]

    @pl.when(i < n_used)
    def _():
        x = xs_ref[...]
        a = lax.dot_general(x, w1_ref[0], (((1,), (1,)), ((), ())),
                            preferred_element_type=jnp.float32)
        b = lax.dot_general(x, w3_ref[0], (((1,), (1,)), ((), ())),
                            preferred_element_type=jnp.float32)
        h = a * (1.0 / (1.0 + jnp.exp(-a))) * b
        o = lax.dot_general(h, w2_ref[0], (((1,), (1,)), ((), ())),
                            preferred_element_type=jnp.float32)
        rows = pl.ds(i * TM, TM)

        @pl.when(j == 0)
        def _():
            out_ref[rows, :] = o

        @pl.when(j > 0)
        def _():
            out_ref[rows, :] += o


def _ffn(tile_expert, xs, w1, w2, w3):
    def iclamp(i, te):
        return jnp.minimum(i, te[NT] - 1)

    grid_spec = pltpu.PrefetchScalarGridSpec(
        num_scalar_prefetch=1,
        grid=(NJ, NT),
        in_specs=[
            pl.BlockSpec((TM, D), lambda j, i, te: (iclamp(i, te), 0)),
            pl.BlockSpec((1, FB, D), lambda j, i, te: (te[iclamp(i, te)], j, 0)),
            pl.BlockSpec((1, FB, D), lambda j, i, te: (te[iclamp(i, te)], j, 0)),
            pl.BlockSpec((1, D, FB), lambda j, i, te: (te[iclamp(i, te)], 0, j)),
        ],
        out_specs=pl.BlockSpec((PMAX, D), lambda j, i, te: (0, 0)),
    )
    return pl.pallas_call(
        _ffn_body,
        grid_spec=grid_spec,
        out_shape=jax.ShapeDtypeStruct((PMAX, D), jnp.float32),
        compiler_params=pltpu.CompilerParams(
            dimension_semantics=("arbitrary", "arbitrary"),
            vmem_limit_bytes=60 * 1024 * 1024,
        ),
    )(tile_expert, xs, w1, w3, w2)


def _combine_body(y_hbm, p0_hbm, p1_hbm, wv0_hbm, wv1_hbm, out_hbm,
                  i0_v, i1_v, wv0_v, wv1_v, buf0, buf1, sem):
    wid = lax.axis_index("s") * 2 + lax.axis_index("c")
    t0 = wid * TOK_W
    pltpu.sync_copy(p0_hbm.at[pl.ds(t0, TOK_W)], i0_v)
    pltpu.sync_copy(p1_hbm.at[pl.ds(t0, TOK_W)], i1_v)
    pltpu.sync_copy(wv0_hbm.at[pl.ds(t0, TOK_W)], wv0_v)
    pltpu.sync_copy(wv1_hbm.at[pl.ds(t0, TOK_W)], wv1_v)
    c0 = pltpu.async_copy(y_hbm.at[i0_v], buf0, sem)
    c1 = pltpu.async_copy(y_hbm.at[i1_v], buf1, sem)
    c0.wait()
    c1.wait()

    def tok_body(i, _):
        lanes = jnp.zeros((16,), jnp.int32) + i
        w0b = plsc.load_gather(wv0_v, [lanes])
        w1b = plsc.load_gather(wv1_v, [lanes])
        for cc in range(D // 16):
            v = (w0b * buf0[i, pl.ds(cc * 16, 16)]
                 + w1b * buf1[i, pl.ds(cc * 16, 16)])
            buf0[i, pl.ds(cc * 16, 16)] = v
        return 0

    lax.fori_loop(0, TOK_W, tok_body, 0)
    pltpu.sync_copy(buf0, out_hbm.at[pl.ds(t0, TOK_W)])


def _combine(y, p0, p1, w0, w1):
    mesh = plsc.VectorSubcoreMesh(core_axis_name="c", subcore_axis_name="s")
    return pl.kernel(
        _combine_body,
        out_type=jax.ShapeDtypeStruct((N, D), jnp.float32),
        mesh=mesh,
        scratch_types=[
            pltpu.VMEM((TOK_W,), jnp.int32),
            pltpu.VMEM((TOK_W,), jnp.int32),
            pltpu.VMEM((TOK_W,), jnp.float32),
            pltpu.VMEM((TOK_W,), jnp.float32),
            pltpu.VMEM((TOK_W, D), jnp.float32),
            pltpu.VMEM((TOK_W, D), jnp.float32),
            pltpu.SemaphoreType.DMA,
        ],
        compiler_params=pltpu.CompilerParams(needs_layout_passes=False),
    )(y, p0, p1, w0, w1)


def kernel(x, Wr, W1, W2, W3):
    Bx, Sx, Dx = x.shape
    x_flat = x.reshape(N, D)
    wr_pad = jnp.concatenate(
        [Wr, jnp.zeros((EP - E, D), jnp.float32)], axis=0)

    pos, w0, w1, cnt, aux = _router(x_flat, wr_pad)
    pos_flat = pos.reshape(A)
    p0 = pos_flat[:N]
    p1 = pos_flat[N:]
    counts = cnt[0, :E]

    tiles_e = (counts + (TM - 1)) // TM
    ends = jnp.cumsum(tiles_e)
    ti = jnp.arange(NT, dtype=jnp.int32)
    tile_expert = jnp.minimum(
        jnp.sum((ti[:, None] >= ends[None, :]).astype(jnp.int32), axis=1),
        E - 1).astype(jnp.int32)
    tile_expert = jnp.concatenate(
        [tile_expert, ends[E - 1:E].astype(jnp.int32)])

    xs = _dispatch(x_flat, p0, p1)
    y = _ffn(tile_expert, xs, W1, W2, W3)
    out = _combine(y, p0, p1, w0.reshape(N), w1.reshape(N))
    return out.reshape(Bx, Sx, Dx), aux[0, 0]


# speedup vs baseline: 2.1115x; 1.0786x over previous
import functools

import jax
import jax.numpy as jnp
from jax import lax
from jax.experimental import pallas as pl
from jax.experimental.pallas import tpu as pltpu
from jax.experimental.pallas import tpu_sc as plsc

N = 2048
D = 768
F = 3072
E = 8
EP = 128
K = 2
A = N * K
CH = 128
NCH = A // CH
TM = 256
PMAX = ((A + E * (TM - 1)) + TM - 1) // TM * TM
NT = PMAX // TM
FB = 1536
NJ = F // FB
NW = 32
TOK_W = N // NW


def _router_body(x_ref, wr_ref, pos_ref, w0_ref, w1_ref, cnt_ref, aux_ref):
    x = x_ref[...]
    wr = wr_ref[...]
    logits = lax.dot_general(x, wr, (((1,), (1,)), ((), ())),
                             preferred_element_type=jnp.float32)
    lane = lax.broadcasted_iota(jnp.int32, (N, EP), 1)
    valid = lane < E
    logits = jnp.where(valid, logits, -1e30)
    m = jnp.max(logits, axis=1, keepdims=True)
    ex = jnp.exp(logits - m)
    ex = jnp.where(valid, ex, 0.0)
    denom = jnp.sum(ex, axis=1, keepdims=True)
    probs = ex / denom

    p1 = jnp.max(probs, axis=1, keepdims=True)
    i1 = jnp.min(jnp.where(probs == p1, lane, EP), axis=1, keepdims=True)
    probs2 = jnp.where(lane == i1, -1.0, probs)
    p2 = jnp.max(probs2, axis=1, keepdims=True)
    i2 = jnp.min(jnp.where(probs2 == p2, lane, EP), axis=1, keepdims=True)
    s = p1 + p2
    w0_ref[...] = p1 / s
    w1_ref[...] = p2 / s

    e1 = (lane == i1).astype(jnp.float32)
    e2 = (lane == i2).astype(jnp.float32)
    onehot = jnp.concatenate([e1, e2], axis=0)

    r = lax.broadcasted_iota(jnp.int32, (CH, CH), 0)
    c = lax.broadcasted_iota(jnp.int32, (CH, CH), 1)
    ls = (r > c).astype(jnp.float32)
    us = (r < c).astype(jnp.float32)
    cums = []
    tots = []
    for ci in range(NCH):
        oc = onehot[ci * CH:(ci + 1) * CH, :]
        cums.append(lax.dot_general(ls, oc, (((1,), (0,)), ((), ())),
                                    preferred_element_type=jnp.float32))
        tots.append(jnp.sum(oc, axis=0, keepdims=True))
    t = jnp.concatenate(tots, axis=0)
    r32 = lax.broadcasted_iota(jnp.int32, (NCH, NCH), 0)
    c32 = lax.broadcasted_iota(jnp.int32, (NCH, NCH), 1)
    ls32 = (r32 > c32).astype(jnp.float32)
    off = lax.dot_general(ls32, t, (((1,), (0,)), ((), ())),
                          preferred_element_type=jnp.float32)
    counts = jnp.sum(t, axis=0, keepdims=True)
    padded = jnp.floor((counts + (TM - 1)) / TM) * TM
    base = lax.dot_general(padded, us, (((1,), (0,)), ((), ())),
                           preferred_element_type=jnp.float32)

    poss = []
    for ci in range(NCH):
        oc = onehot[ci * CH:(ci + 1) * CH, :]
        pc = cums[ci] + off[ci:ci + 1, :] + base
        poss.append(jnp.sum(oc * pc, axis=1, keepdims=True))
    pos = jnp.concatenate(poss, axis=0)
    pos_ref[...] = pos.astype(jnp.int32)
    cnt_ref[...] = counts.astype(jnp.int32)

    importance = jnp.sum(probs, axis=0, keepdims=True)
    aux = jnp.sum(importance * counts) * (E / (N * N + 1e-06))
    aux_ref[...] = jnp.broadcast_to(aux, (1, 1))


def _router(x_flat, wr_pad):
    return pl.pallas_call(
        _router_body,
        out_shape=(
            jax.ShapeDtypeStruct((A, 1), jnp.int32),
            jax.ShapeDtypeStruct((N, 1), jnp.float32),
            jax.ShapeDtypeStruct((N, 1), jnp.float32),
            jax.ShapeDtypeStruct((1, EP), jnp.int32),
            jax.ShapeDtypeStruct((1, 1), jnp.float32),
        ),
    )(x_flat, wr_pad)


def _dispatch_body(x_hbm, p0_hbm, p1_hbm, xs_hbm, i0_v, i1_v, xbuf, sem):
    wid = lax.axis_index("s") * 2 + lax.axis_index("c")
    t0 = wid * TOK_W
    pltpu.sync_copy(p0_hbm.at[pl.ds(t0, TOK_W)], i0_v)
    pltpu.sync_copy(p1_hbm.at[pl.ds(t0, TOK_W)], i1_v)
    pltpu.sync_copy(x_hbm.at[pl.ds(t0, TOK_W)], xbuf)
    pltpu.async_copy(xbuf, xs_hbm.at[i0_v], sem).wait()
    pltpu.async_copy(xbuf, xs_hbm.at[i1_v], sem).wait()


def _dispatch(x_flat, p0, p1):
    mesh = plsc.VectorSubcoreMesh(core_axis_name="c", subcore_axis_name="s")
    return pl.kernel(
        _dispatch_body,
        out_type=jax.ShapeDtypeStruct((PMAX, D), jnp.float32),
        mesh=mesh,
        scratch_types=[
            pltpu.VMEM((TOK_W,), jnp.int32),
            pltpu.VMEM((TOK_W,), jnp.int32),
            pltpu.VMEM((TOK_W, D), jnp.float32),
            pltpu.SemaphoreType.DMA,
        ],
    )(x_flat, p0, p1)


def _ffn_body(te_ref, xs_ref, w1_ref, w3_ref, w2_ref, out_ref):
    j = pl.program_id(0)
    i = pl.program_id(1)
    n_used = te_ref[NT]

    @pl.when(i < n_used)
    def _():
        x = xs_ref[...]
        a = lax.dot_general(x, w1_ref[0], (((1,), (1,)), ((), ())),
                            preferred_element_type=jnp.float32)
        b = lax.dot_general(x, w3_ref[0], (((1,), (1,)), ((), ())),
                            preferred_element_type=jnp.float32)
        h = a * (1.0 / (1.0 + jnp.exp(-a))) * b
        o = lax.dot_general(h, w2_ref[0], (((1,), (1,)), ((), ())),
                            preferred_element_type=jnp.float32)
        rows = pl.ds(i * TM, TM)

        @pl.when(j == 0)
        def _():
            out_ref[rows, :] = o

        @pl.when(j > 0)
        def _():
            out_ref[rows, :] += o


def _ffn(tile_expert, xs, w1, w2, w3):
    def iclamp(i, te):
        return jnp.minimum(i, te[NT] - 1)

    grid_spec = pltpu.PrefetchScalarGridSpec(
        num_scalar_prefetch=1,
        grid=(NJ, NT),
        in_specs=[
            pl.BlockSpec((TM, D), lambda j, i, te: (iclamp(i, te), 0)),
            pl.BlockSpec((1, FB, D), lambda j, i, te: (te[iclamp(i, te)], j, 0)),
            pl.BlockSpec((1, FB, D), lambda j, i, te: (te[iclamp(i, te)], j, 0)),
            pl.BlockSpec((1, D, FB), lambda j, i, te: (te[iclamp(i, te)], 0, j)),
        ],
        out_specs=pl.BlockSpec((PMAX, D), lambda j, i, te: (0, 0)),
    )
    return pl.pallas_call(
        _ffn_body,
        grid_spec=grid_spec,
        out_shape=jax.ShapeDtypeStruct((PMAX, D), jnp.float32),
        compiler_params=pltpu.CompilerParams(
            dimension_semantics=("arbitrary", "arbitrary"),
            vmem_limit_bytes=60 * 1024 * 1024,
        ),
    )(tile_expert, xs, w1, w3, w2)


def _combine_body(y_hbm, p0_hbm, p1_hbm, wv0_hbm, wv1_hbm, out_hbm,
                  i0_v, i1_v, wv0_v, wv1_v, buf0, buf1, sem):
    wid = lax.axis_index("s") * 2 + lax.axis_index("c")
    t0 = wid * TOK_W
    pltpu.sync_copy(p0_hbm.at[pl.ds(t0, TOK_W)], i0_v)
    pltpu.sync_copy(p1_hbm.at[pl.ds(t0, TOK_W)], i1_v)
    pltpu.sync_copy(wv0_hbm.at[pl.ds(t0, TOK_W)], wv0_v)
    pltpu.sync_copy(wv1_hbm.at[pl.ds(t0, TOK_W)], wv1_v)
    c0 = pltpu.async_copy(y_hbm.at[i0_v], buf0, sem)
    c1 = pltpu.async_copy(y_hbm.at[i1_v], buf1, sem)
    c0.wait()
    c1.wait()

    def tok_body(i, _):
        lanes = jnp.zeros((16,), jnp.int32) + i
        w0b = plsc.load_gather(wv0_v, [lanes])
        w1b = plsc.load_gather(wv1_v, [lanes])
        for cc in range(D // 16):
            v = (w0b * buf0[i, pl.ds(cc * 16, 16)]
                 + w1b * buf1[i, pl.ds(cc * 16, 16)])
            buf0[i, pl.ds(cc * 16, 16)] = v
        return 0

    lax.fori_loop(0, TOK_W, tok_body, 0)
    pltpu.sync_copy(buf0, out_hbm.at[pl.ds(t0, TOK_W)])


def _combine(y, p0, p1, w0, w1):
    mesh = plsc.VectorSubcoreMesh(core_axis_name="c", subcore_axis_name="s")
    return pl.kernel(
        _combine_body,
        out_type=jax.ShapeDtypeStruct((N, D), jnp.float32),
        mesh=mesh,
        scratch_types=[
            pltpu.VMEM((TOK_W,), jnp.int32),
            pltpu.VMEM((TOK_W,), jnp.int32),
            pltpu.VMEM((TOK_W,), jnp.float32),
            pltpu.VMEM((TOK_W,), jnp.float32),
            pltpu.VMEM((TOK_W, D), jnp.float32),
            pltpu.VMEM((TOK_W, D), jnp.float32),
            pltpu.SemaphoreType.DMA,
        ],
        compiler_params=pltpu.CompilerParams(needs_layout_passes=False),
    )(y, p0, p1, w0, w1)


def kernel(x, Wr, W1, W2, W3):
    Bx, Sx, Dx = x.shape
    x_flat = x.reshape(N, D)
    wr_pad = jnp.concatenate(
        [Wr, jnp.zeros((EP - E, D), jnp.float32)], axis=0)

    pos, w0, w1, cnt, aux = _router(x_flat, wr_pad)
    pos_flat = pos.reshape(A)
    p0 = pos_flat[:N]
    p1 = pos_flat[N:]
    counts = cnt[0, :E]

    tiles_e = (counts + (TM - 1)) // TM
    ends = jnp.cumsum(tiles_e)
    ti = jnp.arange(NT, dtype=jnp.int32)
    tile_expert = jnp.minimum(
        jnp.sum((ti[:, None] >= ends[None, :]).astype(jnp.int32), axis=1),
        E - 1).astype(jnp.int32)
    tile_expert = jnp.concatenate(
        [tile_expert, ends[E - 1:E].astype(jnp.int32)])

    xs = _dispatch(x_flat, p0, p1)
    y = _ffn(tile_expert, xs, W1, W2, W3)
    out = _combine(y, p0, p1, w0.reshape(N), w1.reshape(N))
    return out.reshape(Bx, Sx, Dx), aux[0, 0]


# speedup vs baseline: 2.1124x; 1.0004x over previous
import functools

import jax
import jax.numpy as jnp
from jax import lax
from jax.experimental import pallas as pl
from jax.experimental.pallas import tpu as pltpu
from jax.experimental.pallas import tpu_sc as plsc

N = 2048
D = 768
F = 3072
E = 8
EP = 128
K = 2
A = N * K
CH = 128
NCH = A // CH
TM = 256
PMAX = ((A + E * (TM - 1)) + TM - 1) // TM * TM
NT = PMAX // TM
FB = 1536
NJ = F // FB
NW = 32
TOK_W = N // NW


def _router_body(x_ref, wr_ref, pos_ref, w0_ref, w1_ref, te_ref, aux_ref):
    x = x_ref[...]
    wr = wr_ref[...]
    logits = lax.dot_general(x, wr, (((1,), (1,)), ((), ())),
                             preferred_element_type=jnp.float32)
    lane = lax.broadcasted_iota(jnp.int32, (N, EP), 1)
    valid = lane < E
    logits = jnp.where(valid, logits, -1e30)
    m = jnp.max(logits, axis=1, keepdims=True)
    ex = jnp.exp(logits - m)
    ex = jnp.where(valid, ex, 0.0)
    denom = jnp.sum(ex, axis=1, keepdims=True)
    probs = ex / denom

    p1 = jnp.max(probs, axis=1, keepdims=True)
    i1 = jnp.min(jnp.where(probs == p1, lane, EP), axis=1, keepdims=True)
    probs2 = jnp.where(lane == i1, -1.0, probs)
    p2 = jnp.max(probs2, axis=1, keepdims=True)
    i2 = jnp.min(jnp.where(probs2 == p2, lane, EP), axis=1, keepdims=True)
    s = p1 + p2
    w0_ref[...] = p1 / s
    w1_ref[...] = p2 / s

    e1 = (lane == i1).astype(jnp.float32)
    e2 = (lane == i2).astype(jnp.float32)
    onehot = jnp.concatenate([e1, e2], axis=0)

    r = lax.broadcasted_iota(jnp.int32, (CH, CH), 0)
    c = lax.broadcasted_iota(jnp.int32, (CH, CH), 1)
    ls = (r > c).astype(jnp.float32)
    us = (r < c).astype(jnp.float32)
    cums = []
    tots = []
    for ci in range(NCH):
        oc = onehot[ci * CH:(ci + 1) * CH, :]
        cums.append(lax.dot_general(ls, oc, (((1,), (0,)), ((), ())),
                                    preferred_element_type=jnp.float32))
        tots.append(jnp.sum(oc, axis=0, keepdims=True))
    t = jnp.concatenate(tots, axis=0)
    r32 = lax.broadcasted_iota(jnp.int32, (NCH, NCH), 0)
    c32 = lax.broadcasted_iota(jnp.int32, (NCH, NCH), 1)
    ls32 = (r32 > c32).astype(jnp.float32)
    off = lax.dot_general(ls32, t, (((1,), (0,)), ((), ())),
                          preferred_element_type=jnp.float32)
    counts = jnp.sum(t, axis=0, keepdims=True)
    padded = jnp.floor((counts + (TM - 1)) / TM) * TM
    base = lax.dot_general(padded, us, (((1,), (0,)), ((), ())),
                           preferred_element_type=jnp.float32)

    poss = []
    for ci in range(NCH):
        oc = onehot[ci * CH:(ci + 1) * CH, :]
        pc = cums[ci] + off[ci:ci + 1, :] + base
        poss.append(jnp.sum(oc * pc, axis=1, keepdims=True))
    pos = jnp.concatenate(poss, axis=0)
    pos_ref[...] = pos.astype(jnp.int32)

    ntiles = padded / TM
    ends = lax.dot_general(ntiles, (r <= c).astype(jnp.float32),
                           (((1,), (0,)), ((), ())),
                           preferred_element_type=jnp.float32)
    dends = jnp.where(r == c, jnp.broadcast_to(ends, (CH, CH)), 0.0)
    ends_mat = lax.dot_general(dends, jnp.ones((CH, CH), jnp.float32),
                               (((1,), (0,)), ((), ())),
                               preferred_element_type=jnp.float32)
    lanes_f = c.astype(jnp.float32)
    m_te = jnp.logical_and(ends_mat <= lanes_f,
                           lax.broadcasted_iota(jnp.int32, (CH, CH), 0) < E)
    te_row = jnp.sum(m_te.astype(jnp.float32), axis=0, keepdims=True)
    te_row = jnp.minimum(te_row, E - 1.0)
    n_used = jnp.sum(jnp.where(lane[:1, :] == E - 1, ends, 0.0))
    te_row = jnp.where(lane[:1, :] == NT, n_used, te_row)
    te_ref[...] = te_row.astype(jnp.int32)

    importance = jnp.sum(probs, axis=0, keepdims=True)
    aux = jnp.sum(importance * counts) * (E / (N * N + 1e-06))
    aux_ref[...] = jnp.broadcast_to(aux, (1, 1))


def _router(x_flat, wr_pad):
    return pl.pallas_call(
        _router_body,
        out_shape=(
            jax.ShapeDtypeStruct((A, 1), jnp.int32),
            jax.ShapeDtypeStruct((N, 1), jnp.float32),
            jax.ShapeDtypeStruct((N, 1), jnp.float32),
            jax.ShapeDtypeStruct((1, EP), jnp.int32),
            jax.ShapeDtypeStruct((1, 1), jnp.float32),
        ),
    )(x_flat, wr_pad)


def _dispatch_body(x_hbm, p0_hbm, p1_hbm, xs_hbm, i0_v, i1_v, xbuf, sem):
    wid = lax.axis_index("s") * 2 + lax.axis_index("c")
    t0 = wid * TOK_W
    pltpu.sync_copy(p0_hbm.at[pl.ds(t0, TOK_W)], i0_v)
    pltpu.sync_copy(p1_hbm.at[pl.ds(t0, TOK_W)], i1_v)
    pltpu.sync_copy(x_hbm.at[pl.ds(t0, TOK_W)], xbuf)
    pltpu.async_copy(xbuf, xs_hbm.at[i0_v], sem).wait()
    pltpu.async_copy(xbuf, xs_hbm.at[i1_v], sem).wait()


def _dispatch(x_flat, p0, p1):
    mesh = plsc.VectorSubcoreMesh(core_axis_name="c", subcore_axis_name="s")
    return pl.kernel(
        _dispatch_body,
        out_type=jax.ShapeDtypeStruct((PMAX, D), jnp.float32),
        mesh=mesh,
        scratch_types=[
            pltpu.VMEM((TOK_W,), jnp.int32),
            pltpu.VMEM((TOK_W,), jnp.int32),
            pltpu.VMEM((TOK_W, D), jnp.float32),
            pltpu.SemaphoreType.DMA,
        ],
    )(x_flat, p0, p1)


def _ffn_body(te_ref, xs_ref, w1_ref, w3_ref, w2_ref, out_ref):
    j = pl.program_id(0)
    i = pl.program_id(1)
    n_used = te_ref[NT]

    @pl.when(i < n_used)
    def _():
        x = xs_ref[...]
        a = lax.dot_general(x, w1_ref[0], (((1,), (1,)), ((), ())),
                            preferred_element_type=jnp.float32)
        b = lax.dot_general(x, w3_ref[0], (((1,), (1,)), ((), ())),
                            preferred_element_type=jnp.float32)
        h = a * (1.0 / (1.0 + jnp.exp(-a))) * b
        o = lax.dot_general(h, w2_ref[0], (((1,), (1,)), ((), ())),
                            preferred_element_type=jnp.float32)
        rows = pl.ds(i * TM, TM)

        @pl.when(j == 0)
        def _():
            out_ref[rows, :] = o

        @pl.when(j > 0)
        def _():
            out_ref[rows, :] += o


def _ffn(tile_expert, xs, w1, w2, w3):
    def iclamp(i, te):
        return jnp.minimum(i, te[NT] - 1)

    grid_spec = pltpu.PrefetchScalarGridSpec(
        num_scalar_prefetch=1,
        grid=(NJ, NT),
        in_specs=[
            pl.BlockSpec((TM, D), lambda j, i, te: (iclamp(i, te), 0)),
            pl.BlockSpec((1, FB, D), lambda j, i, te: (te[iclamp(i, te)], j, 0)),
            pl.BlockSpec((1, FB, D), lambda j, i, te: (te[iclamp(i, te)], j, 0)),
            pl.BlockSpec((1, D, FB), lambda j, i, te: (te[iclamp(i, te)], 0, j)),
        ],
        out_specs=pl.BlockSpec((PMAX, D), lambda j, i, te: (0, 0)),
    )
    return pl.pallas_call(
        _ffn_body,
        grid_spec=grid_spec,
        out_shape=jax.ShapeDtypeStruct((PMAX, D), jnp.float32),
        compiler_params=pltpu.CompilerParams(
            dimension_semantics=("arbitrary", "arbitrary"),
            vmem_limit_bytes=60 * 1024 * 1024,
        ),
    )(tile_expert, xs, w1, w3, w2)


def _combine_body(y_hbm, p0_hbm, p1_hbm, wv0_hbm, wv1_hbm, out_hbm,
                  i0_v, i1_v, wv0_v, wv1_v, buf0, buf1, sem):
    wid = lax.axis_index("s") * 2 + lax.axis_index("c")
    t0 = wid * TOK_W
    pltpu.sync_copy(p0_hbm.at[pl.ds(t0, TOK_W)], i0_v)
    pltpu.sync_copy(p1_hbm.at[pl.ds(t0, TOK_W)], i1_v)
    pltpu.sync_copy(wv0_hbm.at[pl.ds(t0, TOK_W)], wv0_v)
    pltpu.sync_copy(wv1_hbm.at[pl.ds(t0, TOK_W)], wv1_v)
    c0 = pltpu.async_copy(y_hbm.at[i0_v], buf0, sem)
    c1 = pltpu.async_copy(y_hbm.at[i1_v], buf1, sem)
    c0.wait()
    c1.wait()

    def tok_body(i, _):
        lanes = jnp.zeros((16,), jnp.int32) + i
        w0b = plsc.load_gather(wv0_v, [lanes])
        w1b = plsc.load_gather(wv1_v, [lanes])
        for cc in range(D // 16):
            v = (w0b * buf0[i, pl.ds(cc * 16, 16)]
                 + w1b * buf1[i, pl.ds(cc * 16, 16)])
            buf0[i, pl.ds(cc * 16, 16)] = v
        return 0

    lax.fori_loop(0, TOK_W, tok_body, 0)
    pltpu.sync_copy(buf0, out_hbm.at[pl.ds(t0, TOK_W)])


def _combine(y, p0, p1, w0, w1):
    mesh = plsc.VectorSubcoreMesh(core_axis_name="c", subcore_axis_name="s")
    return pl.kernel(
        _combine_body,
        out_type=jax.ShapeDtypeStruct((N, D), jnp.float32),
        mesh=mesh,
        scratch_types=[
            pltpu.VMEM((TOK_W,), jnp.int32),
            pltpu.VMEM((TOK_W,), jnp.int32),
            pltpu.VMEM((TOK_W,), jnp.float32),
            pltpu.VMEM((TOK_W,), jnp.float32),
            pltpu.VMEM((TOK_W, D), jnp.float32),
            pltpu.VMEM((TOK_W, D), jnp.float32),
            pltpu.SemaphoreType.DMA,
        ],
        compiler_params=pltpu.CompilerParams(needs_layout_passes=False),
    )(y, p0, p1, w0, w1)


def kernel(x, Wr, W1, W2, W3):
    Bx, Sx, Dx = x.shape
    x_flat = x.reshape(N, D)
    wr_pad = jnp.concatenate(
        [Wr, jnp.zeros((EP - E, D), jnp.float32)], axis=0)

    pos, w0, w1, te, aux = _router(x_flat, wr_pad)
    pos_flat = pos.reshape(A)
    p0 = pos_flat[:N]
    p1 = pos_flat[N:]
    tile_expert = te.reshape(EP)

    xs = _dispatch(x_flat, p0, p1)
    y = _ffn(tile_expert, xs, W1, W2, W3)
    out = _combine(y, p0, p1, w0.reshape(N), w1.reshape(N))
    return out.reshape(Bx, Sx, Dx), aux[0, 0]


# speedup vs baseline: 2.3051x; 1.0912x over previous
import functools

import jax
import jax.numpy as jnp
from jax import lax
from jax.experimental import pallas as pl
from jax.experimental.pallas import tpu as pltpu
from jax.experimental.pallas import tpu_sc as plsc

N = 2048
D = 768
F = 3072
E = 8
EP = 128
K = 2
A = N * K
CH = 128
NCH = A // CH
TM = 512
PMAX = ((A + E * (TM - 1)) + TM - 1) // TM * TM
NT = PMAX // TM
FB = 1536
NJ = F // FB
NW = 32
TOK_W = N // NW


def _router_body(x_ref, wr_ref, pos_ref, w0_ref, w1_ref, te_ref, aux_ref):
    x = x_ref[...]
    wr = wr_ref[...]
    logits = lax.dot_general(x, wr, (((1,), (1,)), ((), ())),
                             preferred_element_type=jnp.float32)
    lane = lax.broadcasted_iota(jnp.int32, (N, EP), 1)
    valid = lane < E
    logits = jnp.where(valid, logits, -1e30)
    m = jnp.max(logits, axis=1, keepdims=True)
    ex = jnp.exp(logits - m)
    ex = jnp.where(valid, ex, 0.0)
    denom = jnp.sum(ex, axis=1, keepdims=True)
    probs = ex / denom

    p1 = jnp.max(probs, axis=1, keepdims=True)
    i1 = jnp.min(jnp.where(probs == p1, lane, EP), axis=1, keepdims=True)
    probs2 = jnp.where(lane == i1, -1.0, probs)
    p2 = jnp.max(probs2, axis=1, keepdims=True)
    i2 = jnp.min(jnp.where(probs2 == p2, lane, EP), axis=1, keepdims=True)
    s = p1 + p2
    w0_ref[...] = p1 / s
    w1_ref[...] = p2 / s

    e1 = (lane == i1).astype(jnp.float32)
    e2 = (lane == i2).astype(jnp.float32)
    onehot = jnp.concatenate([e1, e2], axis=0)

    r = lax.broadcasted_iota(jnp.int32, (CH, CH), 0)
    c = lax.broadcasted_iota(jnp.int32, (CH, CH), 1)
    ls = (r > c).astype(jnp.float32)
    us = (r < c).astype(jnp.float32)
    cums = []
    tots = []
    for ci in range(NCH):
        oc = onehot[ci * CH:(ci + 1) * CH, :]
        cums.append(lax.dot_general(ls, oc, (((1,), (0,)), ((), ())),
                                    preferred_element_type=jnp.float32))
        tots.append(jnp.sum(oc, axis=0, keepdims=True))
    t = jnp.concatenate(tots, axis=0)
    r32 = lax.broadcasted_iota(jnp.int32, (NCH, NCH), 0)
    c32 = lax.broadcasted_iota(jnp.int32, (NCH, NCH), 1)
    ls32 = (r32 > c32).astype(jnp.float32)
    off = lax.dot_general(ls32, t, (((1,), (0,)), ((), ())),
                          preferred_element_type=jnp.float32)
    counts = jnp.sum(t, axis=0, keepdims=True)
    padded = jnp.floor((counts + (TM - 1)) / TM) * TM
    base = lax.dot_general(padded, us, (((1,), (0,)), ((), ())),
                           preferred_element_type=jnp.float32)

    poss = []
    for ci in range(NCH):
        oc = onehot[ci * CH:(ci + 1) * CH, :]
        pc = cums[ci] + off[ci:ci + 1, :] + base
        poss.append(jnp.sum(oc * pc, axis=1, keepdims=True))
    pos = jnp.concatenate(poss, axis=0)
    pos_ref[...] = pos.astype(jnp.int32)

    ntiles = padded / TM
    ends = lax.dot_general(ntiles, (r <= c).astype(jnp.float32),
                           (((1,), (0,)), ((), ())),
                           preferred_element_type=jnp.float32)
    dends = jnp.where(r == c, jnp.broadcast_to(ends, (CH, CH)), 0.0)
    ends_mat = lax.dot_general(dends, jnp.ones((CH, CH), jnp.float32),
                               (((1,), (0,)), ((), ())),
                               preferred_element_type=jnp.float32)
    lanes_f = c.astype(jnp.float32)
    m_te = jnp.logical_and(ends_mat <= lanes_f,
                           lax.broadcasted_iota(jnp.int32, (CH, CH), 0) < E)
    te_row = jnp.sum(m_te.astype(jnp.float32), axis=0, keepdims=True)
    te_row = jnp.minimum(te_row, E - 1.0)
    n_used = jnp.sum(jnp.where(lane[:1, :] == E - 1, ends, 0.0))
    te_row = jnp.where(lane[:1, :] == NT, n_used, te_row)
    te_ref[...] = te_row.astype(jnp.int32)

    importance = jnp.sum(probs, axis=0, keepdims=True)
    aux = jnp.sum(importance * counts) * (E / (N * N + 1e-06))
    aux_ref[...] = jnp.broadcast_to(aux, (1, 1))


def _router(x_flat, wr_pad):
    return pl.pallas_call(
        _router_body,
        out_shape=(
            jax.ShapeDtypeStruct((A, 1), jnp.int32),
            jax.ShapeDtypeStruct((N, 1), jnp.float32),
            jax.ShapeDtypeStruct((N, 1), jnp.float32),
            jax.ShapeDtypeStruct((1, EP), jnp.int32),
            jax.ShapeDtypeStruct((1, 1), jnp.float32),
        ),
    )(x_flat, wr_pad)


def _dispatch_body(x_hbm, p0_hbm, p1_hbm, xs_hbm, i0_v, i1_v, xbuf, sem):
    wid = lax.axis_index("s") * 2 + lax.axis_index("c")
    t0 = wid * TOK_W
    pltpu.sync_copy(p0_hbm.at[pl.ds(t0, TOK_W)], i0_v)
    pltpu.sync_copy(p1_hbm.at[pl.ds(t0, TOK_W)], i1_v)
    pltpu.sync_copy(x_hbm.at[pl.ds(t0, TOK_W)], xbuf)
    pltpu.async_copy(xbuf, xs_hbm.at[i0_v], sem).wait()
    pltpu.async_copy(xbuf, xs_hbm.at[i1_v], sem).wait()


def _dispatch(x_flat, p0, p1):
    mesh = plsc.VectorSubcoreMesh(core_axis_name="c", subcore_axis_name="s")
    return pl.kernel(
        _dispatch_body,
        out_type=jax.ShapeDtypeStruct((PMAX, D), jnp.float32),
        mesh=mesh,
        scratch_types=[
            pltpu.VMEM((TOK_W,), jnp.int32),
            pltpu.VMEM((TOK_W,), jnp.int32),
            pltpu.VMEM((TOK_W, D), jnp.float32),
            pltpu.SemaphoreType.DMA,
        ],
    )(x_flat, p0, p1)


def _ffn_body(te_ref, xs_ref, w1_ref, w3_ref, w2_ref, out_ref):
    j = pl.program_id(0)
    i = pl.program_id(1)
    n_used = te_ref[NT]

    @pl.when(i < n_used)
    def _():
        x = xs_ref[...]
        a = lax.dot_general(x, w1_ref[0], (((1,), (1,)), ((), ())),
                            preferred_element_type=jnp.float32)
        b = lax.dot_general(x, w3_ref[0], (((1,), (1,)), ((), ())),
                            preferred_element_type=jnp.float32)
        h = a * (1.0 / (1.0 + jnp.exp(-a))) * b
        o = lax.dot_general(h, w2_ref[0], (((1,), (1,)), ((), ())),
                            preferred_element_type=jnp.float32)
        rows = pl.ds(i * TM, TM)

        @pl.when(j == 0)
        def _():
            out_ref[rows, :] = o

        @pl.when(j > 0)
        def _():
            out_ref[rows, :] += o


def _ffn(tile_expert, xs, w1, w2, w3):
    def iclamp(i, te):
        return jnp.minimum(i, te[NT] - 1)

    grid_spec = pltpu.PrefetchScalarGridSpec(
        num_scalar_prefetch=1,
        grid=(NJ, NT),
        in_specs=[
            pl.BlockSpec((TM, D), lambda j, i, te: (iclamp(i, te), 0)),
            pl.BlockSpec((1, FB, D), lambda j, i, te: (te[iclamp(i, te)], j, 0)),
            pl.BlockSpec((1, FB, D), lambda j, i, te: (te[iclamp(i, te)], j, 0)),
            pl.BlockSpec((1, D, FB), lambda j, i, te: (te[iclamp(i, te)], 0, j)),
        ],
        out_specs=pl.BlockSpec((PMAX, D), lambda j, i, te: (0, 0)),
    )
    return pl.pallas_call(
        _ffn_body,
        grid_spec=grid_spec,
        out_shape=jax.ShapeDtypeStruct((PMAX, D), jnp.float32),
        compiler_params=pltpu.CompilerParams(
            dimension_semantics=("arbitrary", "arbitrary"),
            vmem_limit_bytes=60 * 1024 * 1024,
        ),
    )(tile_expert, xs, w1, w3, w2)


def _combine_body(y_hbm, p0_hbm, p1_hbm, wv0_hbm, wv1_hbm, out_hbm,
                  i0_v, i1_v, wv0_v, wv1_v, buf0, buf1, sem):
    wid = lax.axis_index("s") * 2 + lax.axis_index("c")
    t0 = wid * TOK_W
    pltpu.sync_copy(p0_hbm.at[pl.ds(t0, TOK_W)], i0_v)
    pltpu.sync_copy(p1_hbm.at[pl.ds(t0, TOK_W)], i1_v)
    pltpu.sync_copy(wv0_hbm.at[pl.ds(t0, TOK_W)], wv0_v)
    pltpu.sync_copy(wv1_hbm.at[pl.ds(t0, TOK_W)], wv1_v)
    c0 = pltpu.async_copy(y_hbm.at[i0_v], buf0, sem)
    c1 = pltpu.async_copy(y_hbm.at[i1_v], buf1, sem)
    c0.wait()
    c1.wait()

    def tok_body(i, _):
        lanes = jnp.zeros((16,), jnp.int32) + i
        w0b = plsc.load_gather(wv0_v, [lanes])
        w1b = plsc.load_gather(wv1_v, [lanes])
        for cc in range(D // 16):
            v = (w0b * buf0[i, pl.ds(cc * 16, 16)]
                 + w1b * buf1[i, pl.ds(cc * 16, 16)])
            buf0[i, pl.ds(cc * 16, 16)] = v
        return 0

    lax.fori_loop(0, TOK_W, tok_body, 0)
    pltpu.sync_copy(buf0, out_hbm.at[pl.ds(t0, TOK_W)])


def _combine(y, p0, p1, w0, w1):
    mesh = plsc.VectorSubcoreMesh(core_axis_name="c", subcore_axis_name="s")
    return pl.kernel(
        _combine_body,
        out_type=jax.ShapeDtypeStruct((N, D), jnp.float32),
        mesh=mesh,
        scratch_types=[
            pltpu.VMEM((TOK_W,), jnp.int32),
            pltpu.VMEM((TOK_W,), jnp.int32),
            pltpu.VMEM((TOK_W,), jnp.float32),
            pltpu.VMEM((TOK_W,), jnp.float32),
            pltpu.VMEM((TOK_W, D), jnp.float32),
            pltpu.VMEM((TOK_W, D), jnp.float32),
            pltpu.SemaphoreType.DMA,
        ],
        compiler_params=pltpu.CompilerParams(needs_layout_passes=False),
    )(y, p0, p1, w0, w1)


def kernel(x, Wr, W1, W2, W3):
    Bx, Sx, Dx = x.shape
    x_flat = x.reshape(N, D)
    wr_pad = jnp.concatenate(
        [Wr, jnp.zeros((EP - E, D), jnp.float32)], axis=0)

    pos, w0, w1, te, aux = _router(x_flat, wr_pad)
    pos_flat = pos.reshape(A)
    p0 = pos_flat[:N]
    p1 = pos_flat[N:]
    tile_expert = te.reshape(EP)

    xs = _dispatch(x_flat, p0, p1)
    y = _ffn(tile_expert, xs, W1, W2, W3)
    out = _combine(y, p0, p1, w0.reshape(N), w1.reshape(N))
    return out.reshape(Bx, Sx, Dx), aux[0, 0]
